# baseline (speedup 1.0000x reference)
import functools

import jax
import jax.numpy as jnp
from jax import lax
from jax.experimental import pallas as pl
from jax.experimental.pallas import tpu as pltpu
from jax.experimental.pallas import tpu_sc as plsc

L = 200
D = 64
OUT = 2
NW = 16
TPW = 13


def _dan_kernel(win_hbm, sub_hbm, cnt_hbm, t3_hbm, w_hbm, b_hbm, out_hbm,
                win_v, sub_v, cnt_v, wnd_v, st_v, part_sh, red_v,
                w_v, b_v, out_v, sem):
    cid = lax.axis_index("c")
    sid = lax.axis_index("s")

    @pl.when(cid == 0)
    def _():
        pltpu.sync_copy(win_hbm.at[sid], win_v)
        pltpu.sync_copy(sub_hbm.at[sid], sub_v)
        pltpu.sync_copy(cnt_hbm.at[sid], cnt_v)
        winv = win_v[...]
        subv = sub_v[...]
        cntv = cnt_v[...]

        lanes = lax.iota(jnp.int32, 16)
        d0 = lanes >> 3
        d1 = lanes & 7
        acc = [jnp.zeros((16,), jnp.float32) for _ in range(4)]
        for t in range(TPW):
            w = winv[t]
            s = subv[t]
            cw = jnp.full((16,), cntv[t], jnp.float32)
            off = pl.multiple_of(w * 128, 128)
            cps = [
                pltpu.async_copy(
                    t3_hbm.at[r, :, pl.ds(off, 128)], wnd_v.at[r], sem)
                for r in range(8)
            ]
            for cp in cps:
                cp.wait()
            sv = jnp.full((16,), s, jnp.int32)
            for c in range(4):
                g = plsc.load_gather(wnd_v, [2 * c + d0, d1, sv])
                acc[c] = acc[c] + cw * g

        for c in range(4):
            st_v[pl.ds(16 * c, 16)] = acc[c]
        pltpu.sync_copy(st_v, part_sh.at[sid])
        plsc.subcore_barrier()

        @pl.when(sid == 0)
        def _():
            pltpu.sync_copy(w_hbm, w_v)
            pltpu.sync_copy(b_hbm, b_v)
            pltpu.sync_copy(part_sh, red_v)
            tot = [jnp.zeros((16,), jnp.float32) for _ in range(4)]
            for wk in range(NW):
                for c in range(4):
                    tot[c] = tot[c] + red_v[wk, pl.ds(16 * c, 16)]

            inv_n = jnp.float32(1.0 / L)
            m = [a * inv_n for a in tot]
            p0 = (w_v[0, pl.ds(0, 16)] * m[0] + w_v[0, pl.ds(16, 16)] * m[1]
                  + w_v[0, pl.ds(32, 16)] * m[2]
                  + w_v[0, pl.ds(48, 16)] * m[3])
            p1 = (w_v[1, pl.ds(0, 16)] * m[0] + w_v[1, pl.ds(16, 16)] * m[1]
                  + w_v[1, pl.ds(32, 16)] * m[2]
                  + w_v[1, pl.ds(48, 16)] * m[3])

            oh0 = lanes == 0
            oh01 = lanes < OUT
            s0 = jnp.full((16,), jnp.sum(p0), jnp.float32)
            s1 = jnp.full((16,), jnp.sum(p1), jnp.float32)
            bv = b_v[...]
            lv = jnp.where(oh01, jnp.where(oh0, s0, s1) + bv,
                           jnp.float32(-100.0))
            mx = jnp.full(
                (16,),
                jnp.max(jnp.where(oh01, lv, jnp.float32(-1e30))),
                jnp.float32)
            dv = lv - mx
            ev = jnp.exp(dv)
            sv2 = jnp.full((16,), jnp.sum(ev), jnp.float32)
            tt = sv2 - 1.0
            y = tt * (1.0 - tt * (0.5 - tt * (1.0 / 3.0 - 0.25 * tt)))
            for _ in range(3):
                y = y + sv2 * jnp.exp(-y) - 1.0
            out_v[...] = jnp.where(oh01, dv - y, jnp.float32(0.0))
            pltpu.sync_copy(out_v, out_hbm)


@jax.jit
def _dan_call(win, sub, cnt, t3, w, bpad):
    scmesh = plsc.VectorSubcoreMesh(core_axis_name="c", subcore_axis_name="s")
    f = functools.partial(
        pl.kernel,
        out_type=jax.ShapeDtypeStruct((16,), jnp.float32),
        mesh=scmesh,
        compiler_params=pltpu.CompilerParams(needs_layout_passes=False),
        scratch_types=[
            pltpu.VMEM((16,), jnp.int32),
            pltpu.VMEM((16,), jnp.int32),
            pltpu.VMEM((16,), jnp.float32),
            pltpu.VMEM((8, 8, 128), jnp.float32),
            pltpu.VMEM((64,), jnp.float32),
            pltpu.VMEM_SHARED((NW, 64), jnp.float32),
            pltpu.VMEM((NW, 64), jnp.float32),
            pltpu.VMEM((OUT, D), jnp.float32),
            pltpu.VMEM((16,), jnp.float32),
            pltpu.VMEM((16,), jnp.float32),
            pltpu.SemaphoreType.DMA,
        ],
    )(_dan_kernel)
    return f(win, sub, cnt, t3, w, bpad)


def kernel(x, table, W, b):
    row = jnp.arange(NW)[:, None]
    col = jnp.arange(16)[None, :]
    m = row * TPW + col
    valid = (col < TPW) & (m < L)
    mc = jnp.where(valid, m, 0)
    xg = x[mc]
    win = jnp.where(valid, xg >> 7, 0).astype(jnp.int32)
    sub = jnp.where(valid, xg & 127, 0).astype(jnp.int32)
    cnt = valid.astype(jnp.float32)
    t3 = table.T.reshape(8, 8, 1000000)
    bpad = jnp.zeros((16,), jnp.float32).at[:OUT].set(b)
    res = _dan_call(win, sub, cnt, t3, W, bpad)
    return res[:OUT]


# speedup vs baseline: 1.1494x; 1.1494x over previous
import functools

import jax
import jax.numpy as jnp
from jax import lax
from jax.experimental import pallas as pl
from jax.experimental.pallas import tpu as pltpu
from jax.experimental.pallas import tpu_sc as plsc

L = 200
D = 64
OUT = 2
NW = 16
TPW = 13


def _dan_kernel(win_hbm, sub_hbm, cnt_hbm, t3_hbm, w_hbm, b_hbm, out_hbm,
                win_v, sub_v, cnt_v, wnd_v, st_v, part_sh, red_v,
                w_v, b_v, out_v, sem0, sem1):
    sems = (sem0, sem1)
    cid = lax.axis_index("c")
    sid = lax.axis_index("s")

    @pl.when(cid == 0)
    def _():
        pltpu.sync_copy(win_hbm.at[sid], win_v)
        pltpu.sync_copy(sub_hbm.at[sid], sub_v)
        pltpu.sync_copy(cnt_hbm.at[sid], cnt_v)
        winv = win_v[...]
        subv = sub_v[...]
        cntv = cnt_v[...]

        lanes = lax.iota(jnp.int32, 16)
        d0 = lanes >> 3
        d1 = lanes & 7
        acc = [jnp.zeros((16,), jnp.float32) for _ in range(4)]

        def fire(t):
            b = t % 2
            off = pl.multiple_of(winv[t] * 128, 128)
            return [
                pltpu.async_copy(
                    t3_hbm.at[r, :, pl.ds(off, 128)], wnd_v.at[b, r],
                    sems[b])
                for r in range(8)
            ]

        cps = {0: fire(0)}
        for t in range(TPW):
            if t + 1 < TPW:
                cps[t + 1] = fire(t + 1)
            for cp in cps.pop(t):
                cp.wait()
            s = subv[t]
            cw = jnp.full((16,), cntv[t], jnp.float32)
            sv = jnp.full((16,), s, jnp.int32)
            for c in range(4):
                g = plsc.load_gather(
                    wnd_v.at[t % 2], [2 * c + d0, d1, sv])
                acc[c] = acc[c] + cw * g

        for c in range(4):
            st_v[pl.ds(16 * c, 16)] = acc[c]
        pltpu.sync_copy(st_v, part_sh.at[sid])
        plsc.subcore_barrier()

        @pl.when(sid == 0)
        def _():
            pltpu.sync_copy(w_hbm, w_v)
            pltpu.sync_copy(b_hbm, b_v)
            pltpu.sync_copy(part_sh, red_v)
            tot = [jnp.zeros((16,), jnp.float32) for _ in range(4)]
            for wk in range(NW):
                for c in range(4):
                    tot[c] = tot[c] + red_v[wk, pl.ds(16 * c, 16)]

            inv_n = jnp.float32(1.0 / L)
            m = [a * inv_n for a in tot]
            p0 = (w_v[0, pl.ds(0, 16)] * m[0] + w_v[0, pl.ds(16, 16)] * m[1]
                  + w_v[0, pl.ds(32, 16)] * m[2]
                  + w_v[0, pl.ds(48, 16)] * m[3])
            p1 = (w_v[1, pl.ds(0, 16)] * m[0] + w_v[1, pl.ds(16, 16)] * m[1]
                  + w_v[1, pl.ds(32, 16)] * m[2]
                  + w_v[1, pl.ds(48, 16)] * m[3])

            oh0 = lanes == 0
            oh01 = lanes < OUT
            s0 = jnp.full((16,), jnp.sum(p0), jnp.float32)
            s1 = jnp.full((16,), jnp.sum(p1), jnp.float32)
            bv = b_v[...]
            lv = jnp.where(oh01, jnp.where(oh0, s0, s1) + bv,
                           jnp.float32(-100.0))
            mx = jnp.full(
                (16,),
                jnp.max(jnp.where(oh01, lv, jnp.float32(-1e30))),
                jnp.float32)
            dv = lv - mx
            ev = jnp.exp(dv)
            sv2 = jnp.full((16,), jnp.sum(ev), jnp.float32)
            tt = sv2 - 1.0
            y = tt * (1.0 - tt * (0.5 - tt * (1.0 / 3.0 - 0.25 * tt)))
            for _ in range(3):
                y = y + sv2 * jnp.exp(-y) - 1.0
            out_v[...] = jnp.where(oh01, dv - y, jnp.float32(0.0))
            pltpu.sync_copy(out_v, out_hbm)


@jax.jit
def _dan_call(win, sub, cnt, t3, w, bpad):
    scmesh = plsc.VectorSubcoreMesh(core_axis_name="c", subcore_axis_name="s")
    f = functools.partial(
        pl.kernel,
        out_type=jax.ShapeDtypeStruct((16,), jnp.float32),
        mesh=scmesh,
        compiler_params=pltpu.CompilerParams(needs_layout_passes=False),
        scratch_types=[
            pltpu.VMEM((16,), jnp.int32),
            pltpu.VMEM((16,), jnp.int32),
            pltpu.VMEM((16,), jnp.float32),
            pltpu.VMEM((2, 8, 8, 128), jnp.float32),
            pltpu.VMEM((64,), jnp.float32),
            pltpu.VMEM_SHARED((NW, 64), jnp.float32),
            pltpu.VMEM((NW, 64), jnp.float32),
            pltpu.VMEM((OUT, D), jnp.float32),
            pltpu.VMEM((16,), jnp.float32),
            pltpu.VMEM((16,), jnp.float32),
            pltpu.SemaphoreType.DMA,
            pltpu.SemaphoreType.DMA,
        ],
    )(_dan_kernel)
    return f(win, sub, cnt, t3, w, bpad)


def kernel(x, table, W, b):
    row = jnp.arange(NW)[:, None]
    col = jnp.arange(16)[None, :]
    m = row * TPW + col
    valid = (col < TPW) & (m < L)
    mc = jnp.where(valid, m, 0)
    xg = x[mc]
    win = jnp.where(valid, xg >> 7, 0).astype(jnp.int32)
    sub = jnp.where(valid, xg & 127, 0).astype(jnp.int32)
    cnt = valid.astype(jnp.float32)
    t3 = table.T.reshape(8, 8, 1000000)
    bpad = jnp.zeros((16,), jnp.float32).at[:OUT].set(b)
    res = _dan_call(win, sub, cnt, t3, W, bpad)
    return res[:OUT]


# speedup vs baseline: 1.1680x; 1.0162x over previous
import functools

import jax
import jax.numpy as jnp
from jax import lax
from jax.experimental import pallas as pl
from jax.experimental.pallas import tpu as pltpu
from jax.experimental.pallas import tpu_sc as plsc

L = 200
D = 64
OUT = 2
NW = 16
TPW = 13


def _dan_kernel(win_hbm, sub_hbm, cnt_hbm, t2_hbm, w_hbm, b_hbm, out_hbm,
                win_v, sub_v, cnt_v, wnd_v, st_v, part_sh, red_v,
                w_v, b_v, out_v, sem0, sem1):
    sems = (sem0, sem1)
    cid = lax.axis_index("c")
    sid = lax.axis_index("s")

    @pl.when(cid == 0)
    def _():
        pltpu.sync_copy(win_hbm.at[sid], win_v)
        pltpu.sync_copy(sub_hbm.at[sid], sub_v)
        pltpu.sync_copy(cnt_hbm.at[sid], cnt_v)
        winv = win_v[...]
        subv = sub_v[...]
        cntv = cnt_v[...]

        lanes = lax.iota(jnp.int32, 16)
        acc = [jnp.zeros((16,), jnp.float32) for _ in range(4)]

        def fire(t):
            b = t % 2
            off = pl.multiple_of(winv[t] * 128, 128)
            return pltpu.async_copy(
                t2_hbm.at[:, pl.ds(off, 128)], wnd_v.at[b], sems[b])

        cps = {0: fire(0)}
        for t in range(TPW):
            if t + 1 < TPW:
                cps[t + 1] = fire(t + 1)
            cps.pop(t).wait()
            s = subv[t]
            cw = jnp.full((16,), cntv[t], jnp.float32)
            sv = jnp.full((16,), s, jnp.int32)
            for c in range(4):
                g = plsc.load_gather(
                    wnd_v.at[t % 2], [16 * c + lanes, sv])
                acc[c] = acc[c] + cw * g

        for c in range(4):
            st_v[pl.ds(16 * c, 16)] = acc[c]
        pltpu.sync_copy(st_v, part_sh.at[sid])
        plsc.subcore_barrier()

        @pl.when(sid == 0)
        def _():
            pltpu.sync_copy(w_hbm, w_v)
            pltpu.sync_copy(b_hbm, b_v)
            pltpu.sync_copy(part_sh, red_v)
            tot = [jnp.zeros((16,), jnp.float32) for _ in range(4)]
            for wk in range(NW):
                for c in range(4):
                    tot[c] = tot[c] + red_v[wk, pl.ds(16 * c, 16)]

            inv_n = jnp.float32(1.0 / L)
            m = [a * inv_n for a in tot]
            p0 = (w_v[0, pl.ds(0, 16)] * m[0] + w_v[0, pl.ds(16, 16)] * m[1]
                  + w_v[0, pl.ds(32, 16)] * m[2]
                  + w_v[0, pl.ds(48, 16)] * m[3])
            p1 = (w_v[1, pl.ds(0, 16)] * m[0] + w_v[1, pl.ds(16, 16)] * m[1]
                  + w_v[1, pl.ds(32, 16)] * m[2]
                  + w_v[1, pl.ds(48, 16)] * m[3])

            oh0 = lanes == 0
            oh01 = lanes < OUT
            s0 = jnp.full((16,), jnp.sum(p0), jnp.float32)
            s1 = jnp.full((16,), jnp.sum(p1), jnp.float32)
            bv = b_v[...]
            lv = jnp.where(oh01, jnp.where(oh0, s0, s1) + bv,
                           jnp.float32(-100.0))
            mx = jnp.full(
                (16,),
                jnp.max(jnp.where(oh01, lv, jnp.float32(-1e30))),
                jnp.float32)
            dv = lv - mx
            ev = jnp.exp(dv)
            sv2 = jnp.full((16,), jnp.sum(ev), jnp.float32)
            tt = sv2 - 1.0
            y = tt * (1.0 - tt * (0.5 - tt * (1.0 / 3.0 - 0.25 * tt)))
            for _ in range(3):
                y = y + sv2 * jnp.exp(-y) - 1.0
            out_v[...] = jnp.where(oh01, dv - y, jnp.float32(0.0))
            pltpu.sync_copy(out_v, out_hbm)


@jax.jit
def _dan_call(win, sub, cnt, t3, w, bpad):
    scmesh = plsc.VectorSubcoreMesh(core_axis_name="c", subcore_axis_name="s")
    f = functools.partial(
        pl.kernel,
        out_type=jax.ShapeDtypeStruct((16,), jnp.float32),
        mesh=scmesh,
        compiler_params=pltpu.CompilerParams(needs_layout_passes=False),
        scratch_types=[
            pltpu.VMEM((16,), jnp.int32),
            pltpu.VMEM((16,), jnp.int32),
            pltpu.VMEM((16,), jnp.float32),
            pltpu.VMEM((2, 64, 128), jnp.float32),
            pltpu.VMEM((64,), jnp.float32),
            pltpu.VMEM_SHARED((NW, 64), jnp.float32),
            pltpu.VMEM((NW, 64), jnp.float32),
            pltpu.VMEM((OUT, D), jnp.float32),
            pltpu.VMEM((16,), jnp.float32),
            pltpu.VMEM((16,), jnp.float32),
            pltpu.SemaphoreType.DMA,
            pltpu.SemaphoreType.DMA,
        ],
    )(_dan_kernel)
    return f(win, sub, cnt, t3, w, bpad)


def kernel(x, table, W, b):
    row = jnp.arange(NW)[:, None]
    col = jnp.arange(16)[None, :]
    m = row * TPW + col
    valid = (col < TPW) & (m < L)
    mc = jnp.where(valid, m, 0)
    xg = x[mc]
    win = jnp.where(valid, xg >> 7, 0).astype(jnp.int32)
    sub = jnp.where(valid, xg & 127, 0).astype(jnp.int32)
    cnt = valid.astype(jnp.float32)
    t3 = table.T
    bpad = jnp.zeros((16,), jnp.float32).at[:OUT].set(b)
    res = _dan_call(win, sub, cnt, t3, W, bpad)
    return res[:OUT]


# speedup vs baseline: 1.3187x; 1.1290x over previous
import functools

import jax
import jax.numpy as jnp
from jax import lax
from jax.experimental import pallas as pl
from jax.experimental.pallas import tpu as pltpu
from jax.experimental.pallas import tpu_sc as plsc

L = 200
D = 64
OUT = 2
NW = 16
TPW = 13


def _dan_kernel(win_hbm, sub_hbm, cnt_hbm, t2_hbm, w_hbm, b_hbm, out_hbm,
                win_v, sub_v, cnt_v, wnd_v, st_v, part_sh, red_v,
                w_v, b_v, out_v, sem):
    cid = lax.axis_index("c")
    sid = lax.axis_index("s")

    @pl.when(cid == 0)
    def _():
        pltpu.sync_copy(win_hbm.at[sid], win_v)
        pltpu.sync_copy(sub_hbm.at[sid], sub_v)
        pltpu.sync_copy(cnt_hbm.at[sid], cnt_v)
        winv = win_v[...]
        subv = sub_v[...]
        cntv = cnt_v[...]

        lanes = lax.iota(jnp.int32, 16)
        acc = [jnp.zeros((16,), jnp.float32) for _ in range(4)]

        cps = [
            pltpu.async_copy(
                t2_hbm.at[:, pl.ds(pl.multiple_of(winv[t] * 128, 128), 128)],
                wnd_v.at[t], sem)
            for t in range(TPW)
        ]
        for t in range(TPW):
            cps[t].wait()
            s = subv[t]
            cw = jnp.full((16,), cntv[t], jnp.float32)
            sv = jnp.full((16,), s, jnp.int32)
            for c in range(4):
                g = plsc.load_gather(
                    wnd_v.at[t], [16 * c + lanes, sv])
                acc[c] = acc[c] + cw * g

        for c in range(4):
            st_v[pl.ds(16 * c, 16)] = acc[c]
        pltpu.sync_copy(st_v, part_sh.at[sid])
        plsc.subcore_barrier()

        @pl.when(sid == 0)
        def _():
            pltpu.sync_copy(w_hbm, w_v)
            pltpu.sync_copy(b_hbm, b_v)
            pltpu.sync_copy(part_sh, red_v)
            tot = [jnp.zeros((16,), jnp.float32) for _ in range(4)]
            for wk in range(NW):
                for c in range(4):
                    tot[c] = tot[c] + red_v[wk, pl.ds(16 * c, 16)]

            inv_n = jnp.float32(1.0 / L)
            m = [a * inv_n for a in tot]
            p0 = (w_v[0, pl.ds(0, 16)] * m[0] + w_v[0, pl.ds(16, 16)] * m[1]
                  + w_v[0, pl.ds(32, 16)] * m[2]
                  + w_v[0, pl.ds(48, 16)] * m[3])
            p1 = (w_v[1, pl.ds(0, 16)] * m[0] + w_v[1, pl.ds(16, 16)] * m[1]
                  + w_v[1, pl.ds(32, 16)] * m[2]
                  + w_v[1, pl.ds(48, 16)] * m[3])

            oh0 = lanes == 0
            oh01 = lanes < OUT
            s0 = jnp.full((16,), jnp.sum(p0), jnp.float32)
            s1 = jnp.full((16,), jnp.sum(p1), jnp.float32)
            bv = b_v[...]
            lv = jnp.where(oh01, jnp.where(oh0, s0, s1) + bv,
                           jnp.float32(-100.0))
            mx = jnp.full(
                (16,),
                jnp.max(jnp.where(oh01, lv, jnp.float32(-1e30))),
                jnp.float32)
            dv = lv - mx
            ev = jnp.exp(dv)
            sv2 = jnp.full((16,), jnp.sum(ev), jnp.float32)
            tt = sv2 - 1.0
            y = tt * (1.0 - tt * (0.5 - tt * (1.0 / 3.0 - 0.25 * tt)))
            for _ in range(3):
                y = y + sv2 * jnp.exp(-y) - 1.0
            out_v[...] = jnp.where(oh01, dv - y, jnp.float32(0.0))
            pltpu.sync_copy(out_v, out_hbm)


@jax.jit
def _dan_call(win, sub, cnt, t3, w, bpad):
    scmesh = plsc.VectorSubcoreMesh(core_axis_name="c", subcore_axis_name="s")
    f = functools.partial(
        pl.kernel,
        out_type=jax.ShapeDtypeStruct((16,), jnp.float32),
        mesh=scmesh,
        compiler_params=pltpu.CompilerParams(needs_layout_passes=False),
        scratch_types=[
            pltpu.VMEM((16,), jnp.int32),
            pltpu.VMEM((16,), jnp.int32),
            pltpu.VMEM((16,), jnp.float32),
            pltpu.VMEM((TPW, 64, 128), jnp.float32),
            pltpu.VMEM((64,), jnp.float32),
            pltpu.VMEM_SHARED((NW, 64), jnp.float32),
            pltpu.VMEM((NW, 64), jnp.float32),
            pltpu.VMEM((OUT, D), jnp.float32),
            pltpu.VMEM((16,), jnp.float32),
            pltpu.VMEM((16,), jnp.float32),
            pltpu.SemaphoreType.DMA,
        ],
    )(_dan_kernel)
    return f(win, sub, cnt, t3, w, bpad)


def kernel(x, table, W, b):
    row = jnp.arange(NW)[:, None]
    col = jnp.arange(16)[None, :]
    m = row * TPW + col
    valid = (col < TPW) & (m < L)
    mc = jnp.where(valid, m, 0)
    xg = x[mc]
    win = jnp.where(valid, xg >> 7, 0).astype(jnp.int32)
    sub = jnp.where(valid, xg & 127, 0).astype(jnp.int32)
    cnt = valid.astype(jnp.float32)
    t3 = table.T
    bpad = jnp.zeros((16,), jnp.float32).at[:OUT].set(b)
    res = _dan_call(win, sub, cnt, t3, W, bpad)
    return res[:OUT]


# speedup vs baseline: 4.9638x; 3.7640x over previous
import jax
import jax.numpy as jnp
from jax.experimental import pallas as pl
from jax.experimental.pallas import tpu as pltpu

L = 200
D = 64
OUT = 2
WIN = 128
TPB = 40
GRID = L // TPB


def _dan_body(x_sref, *refs):
    blocks = refs[:TPB]
    w_ref, b_ref, out_ref, acc = refs[TPB:]
    j = pl.program_id(0)

    @pl.when(j == 0)
    def _():
        acc[...] = jnp.zeros((D, WIN), jnp.float32)

    lane = jax.lax.broadcasted_iota(jnp.int32, (D, WIN), 1)
    a = acc[...]
    for k in range(TPB):
        sub = x_sref[j * TPB + k] & (WIN - 1)
        a = a + jnp.where(lane == sub, blocks[k][...], jnp.float32(0.0))
    acc[...] = a

    @pl.when(j == GRID - 1)
    def _():
        mean = jnp.sum(a, axis=1) * jnp.float32(1.0 / L)
        logits = w_ref[...] @ mean + b_ref[...]
        mx = jnp.max(logits)
        z = logits - mx
        out_ref[...] = z - jnp.log(jnp.sum(jnp.exp(z)))


def _win_spec(k):
    return pl.BlockSpec((D, WIN), lambda j, xr: (0, xr[j * TPB + k] >> 7))


@jax.jit
def _dan_call(x, t2, w, b):
    return pl.pallas_call(
        _dan_body,
        grid_spec=pltpu.PrefetchScalarGridSpec(
            num_scalar_prefetch=1,
            grid=(GRID,),
            in_specs=[_win_spec(k) for k in range(TPB)]
            + [
                pl.BlockSpec((OUT, D), lambda j, xr: (0, 0)),
                pl.BlockSpec((OUT,), lambda j, xr: (0,)),
            ],
            out_specs=pl.BlockSpec((OUT,), lambda j, xr: (0,)),
            scratch_shapes=[pltpu.VMEM((D, WIN), jnp.float32)],
        ),
        out_shape=jax.ShapeDtypeStruct((OUT,), jnp.float32),
        compiler_params=pltpu.CompilerParams(
            dimension_semantics=("arbitrary",),
        ),
    )(x, *([t2] * TPB), w, b)


def kernel(x, table, W, b):
    t2 = table.T
    return _dan_call(x, t2, W, b)


# speedup vs baseline: 5.0833x; 1.0241x over previous
import jax
import jax.numpy as jnp
from jax.experimental import pallas as pl
from jax.experimental.pallas import tpu as pltpu

L = 200
D = 64
OUT = 2
WIN = 128
TPB = 50
GRID = L // TPB


def _dan_body(x_sref, *refs):
    blocks = refs[:TPB]
    w_ref, b_ref, out_ref, acc = refs[TPB:]
    j = pl.program_id(0)

    @pl.when(j == 0)
    def _():
        acc[...] = jnp.zeros((D, WIN), jnp.float32)

    lane = jax.lax.broadcasted_iota(jnp.int32, (D, WIN), 1)
    a = acc[...]
    for k in range(TPB):
        sub = x_sref[j * TPB + k] & (WIN - 1)
        a = a + jnp.where(lane == sub, blocks[k][...], jnp.float32(0.0))
    acc[...] = a

    @pl.when(j == GRID - 1)
    def _():
        mean = jnp.sum(a, axis=1) * jnp.float32(1.0 / L)
        logits = w_ref[...] @ mean + b_ref[...]
        mx = jnp.max(logits)
        z = logits - mx
        out_ref[...] = z - jnp.log(jnp.sum(jnp.exp(z)))


def _win_spec(k):
    return pl.BlockSpec((D, WIN), lambda j, xr: (0, xr[j * TPB + k] >> 7))


@jax.jit
def _dan_call(x, t2, w, b):
    return pl.pallas_call(
        _dan_body,
        grid_spec=pltpu.PrefetchScalarGridSpec(
            num_scalar_prefetch=1,
            grid=(GRID,),
            in_specs=[_win_spec(k) for k in range(TPB)]
            + [
                pl.BlockSpec((OUT, D), lambda j, xr: (0, 0)),
                pl.BlockSpec((OUT,), lambda j, xr: (0,)),
            ],
            out_specs=pl.BlockSpec((OUT,), lambda j, xr: (0,)),
            scratch_shapes=[pltpu.VMEM((D, WIN), jnp.float32)],
        ),
        out_shape=jax.ShapeDtypeStruct((OUT,), jnp.float32),
        compiler_params=pltpu.CompilerParams(
            dimension_semantics=("arbitrary",),
        ),
    )(x, *([t2] * TPB), w, b)


def kernel(x, table, W, b):
    t2 = table.T
    return _dan_call(x, t2, W, b)
